# baseline (speedup 1.0000x reference)
import functools
import math

import jax
import jax.numpy as jnp
from jax import lax
from jax.experimental import pallas as pl
from jax.experimental.pallas import tpu as pltpu
from jax.experimental.pallas import tpu_sc as plsc

NC = 2
NS = 16
NW = NC * NS
LANES = 16
SPLIT = 96
NBUF = 2


@functools.partial(jax.jit, static_argnums=(2, 3, 4))
def _lookup(xf, table2, b, l, d):
    bat_w = b // NW
    n_chunks = bat_w * 2
    scale = float(math.sqrt(d))
    sizes = (SPLIT, l - SPLIT)
    mesh = plsc.VectorSubcoreMesh(core_axis_name="c", subcore_axis_name="s")

    @functools.partial(
        pl.kernel,
        mesh=mesh,
        out_type=jax.ShapeDtypeStruct((b, l, d), jnp.float32),
        scratch_types=(
            [pltpu.VMEM((bat_w * l,), jnp.int32),
             pltpu.VMEM((112,), jnp.int32)]
            + [pltpu.VMEM((sizes[i % 2], 2 * d), jnp.float32)
               for i in range(NBUF)]
            + [pltpu.VMEM((sizes[i % 2], d), jnp.float32)
               for i in range(NBUF)]
            + [pltpu.SemaphoreType.DMA for _ in range(NBUF)]
            + [pltpu.SemaphoreType.DMA for _ in range(NBUF)]
        ),
    )
    def k(x_hbm, table_hbm, out_hbm, x_v, pair_v, *rest):
        g = rest[:NBUF]
        stage = rest[NBUF:2 * NBUF]
        gsem = rest[2 * NBUF:3 * NBUF]
        ssem = rest[3 * NBUF:4 * NBUF]
        wid = lax.axis_index("s") * NC + lax.axis_index("c")
        base = wid * bat_w
        pltpu.sync_copy(x_hbm.at[wid], x_v)

        def chunk_off(c):
            return (c // 2) * l + (c % 2) * SPLIT

        def prep_pairs(c, size):
            ow = chunk_off(c)
            for o in range(0, size - LANES + 1, LANES):
                pair_v[pl.ds(o, LANES)] = x_v[pl.ds(ow + o, LANES)] >> 1
            if size % LANES:
                o = size - LANES
                pair_v[pl.ds(o, LANES)] = x_v[pl.ds(ow + o, LANES)] >> 1

        def start_gather(bi, size):
            pltpu.async_copy(
                table_hbm.at[pair_v.at[pl.ds(0, size)]], g[bi], gsem[bi])

        def wait_gather(bi):
            pltpu.make_async_copy(
                g[bi], table_hbm.at[pl.ds(0, g[bi].shape[0])],
                gsem[bi]).wait()

        def wait_store(bi, c):
            pltpu.make_async_copy(
                stage[bi],
                out_hbm.at[base].at[
                    pl.ds((c % 2) * SPLIT, stage[bi].shape[0])],
                ssem[bi]).wait()

        def start_store(bi, c):
            pltpu.async_copy(
                stage[bi],
                out_hbm.at[base + c // 2].at[
                    pl.ds((c % 2) * SPLIT, stage[bi].shape[0])],
                ssem[bi])

        def blend(bi, c, size):
            ow = chunk_off(c)

            def do_rows(o, pv, r_lo):
                ps = [jnp.broadcast_to(pv[rr] * scale, (LANES,))
                      for rr in range(r_lo, LANES)]
                for rr in range(r_lo, LANES):
                    r = o + rr
                    for cc in range(d // LANES):
                        left = g[bi][r, pl.ds(cc * LANES, LANES)]
                        right = g[bi][r, pl.ds(d + cc * LANES, LANES)]
                        stage[bi][r, pl.ds(cc * LANES, LANES)] = (
                            left * scale + (right - left) * ps[rr - r_lo])

            def grp_body(gi, c2):
                o = gi * LANES
                pv = (x_v[pl.ds(ow + o, LANES)] & 1).astype(jnp.float32)
                do_rows(o, pv, 0)
                return c2

            lax.fori_loop(0, size // LANES, grp_body, 0)
            if size % LANES:
                tail = size % LANES
                o = size - LANES
                pv = (x_v[pl.ds(ow + o, LANES)] & 1).astype(jnp.float32)
                do_rows(o, pv, LANES - tail)

        prep_pairs(0, sizes[0])
        start_gather(0, sizes[0])

        def group_body(g4, carry):
            for bi in range(NBUF):
                c = g4 * NBUF + bi
                nbi = (bi + 1) % NBUF
                sz = sizes[bi % 2]
                nsz = sizes[(bi + 1) % 2]
                wait_gather(bi)
                if bi == NBUF - 1:
                    @pl.when(g4 < (n_chunks // NBUF) - 1)
                    def _():
                        wait_store(nbi, c + 1)
                        prep_pairs(c + 1, nsz)
                        start_gather(nbi, nsz)
                else:
                    @pl.when(g4 >= 1)
                    def _():
                        wait_store(nbi, c + 1)
                        prep_pairs(c + 1, nsz)
                        start_gather(nbi, nsz)

                    @pl.when(g4 == 0)
                    def _():
                        prep_pairs(c + 1, nsz)
                        start_gather(nbi, nsz)
                blend(bi, c, sz)
                start_store(bi, c)
            return carry

        lax.fori_loop(0, n_chunks // NBUF, group_body, 0)
        for bi in range(NBUF):
            wait_store(bi, bi)

    return k(xf, table2)


def kernel(x, table):
    b, l = x.shape
    _, d = table.shape
    xf = x.astype(jnp.int32).reshape(NW, (b // NW) * l)
    table2 = table.reshape(-1, 2 * d)
    return _lookup(xf, table2, b, l, d)
